# baseline (speedup 1.0000x reference)
import jax
import jax.numpy as jnp
from jax import lax
from jax.experimental import pallas as pl
from jax.experimental.pallas import tpu as pltpu
from jax.experimental.pallas import tpu_sc as plsc

N = 10000
E = 320000
D = 128
SPL = 64
NP = 10240
NC = 2
NS = 16
CHUNK = 128
INNER = 2
EP = 327680
ROWS_PER_S = EP // NS // CHUNK
HROWS = ROWS_PER_S // 2
BLOCKS_H = HROWS // INNER
TSTEPS = BLOCKS_H // 2
RPS = NP // NS


def _make_sc_agg(with_deg):
    out_type = [jax.ShapeDtypeStruct((NC, NP, SPL), jnp.float32)]
    scratch = [
        pltpu.VMEM((HROWS, CHUNK), jnp.int32),
        pltpu.VMEM((HROWS, CHUNK), jnp.int32),
        pltpu.VMEM((INNER * CHUNK, SPL), jnp.float32),
        pltpu.VMEM((INNER * CHUNK, SPL), jnp.float32),
        pltpu.VMEM((INNER * CHUNK, SPL), jnp.bfloat16),
        pltpu.VMEM((INNER * CHUNK, SPL), jnp.bfloat16),
        pltpu.VMEM_SHARED((NP, SPL), jnp.float32),
        pltpu.SemaphoreType.DMA,
        pltpu.SemaphoreType.DMA,
    ]
    if with_deg:
        out_type.append(jax.ShapeDtypeStruct((NC, NP), jnp.float32))
        scratch += [
            pltpu.VMEM((CHUNK,), jnp.float32),
            pltpu.VMEM_SHARED((NP,), jnp.float32),
            pltpu.SemaphoreType.DMA,
        ]

    mesh = plsc.VectorSubcoreMesh(core_axis_name="c", subcore_axis_name="s")

    def body(*refs):
        if with_deg:
            (table3, src2, dst2, zrows, zdeg, out_agg, out_deg,
             src_h, dst_h, rows0, rows1, bf0, bf1, agg_sh, sem_g, sem_s,
             ones_v, deg_sh, sem_d) = refs
        else:
            (table3, src2, dst2, zrows, out_agg,
             src_h, dst_h, rows0, rows1, bf0, bf1, agg_sh,
             sem_g, sem_s) = refs
        c = lax.axis_index("c")
        s = lax.axis_index("s")

        pltpu.sync_copy(zrows, agg_sh.at[pl.ds(s * RPS, RPS)])
        if with_deg:
            pltpu.sync_copy(zdeg, deg_sh.at[pl.ds(s * RPS, RPS)])
            for i in range(CHUNK // 16):
                ones_v[pl.ds(i * 16, 16)] = jnp.ones((16,), jnp.float32)
        plsc.subcore_barrier()

        tbl = table3.at[c]

        def fire_gathers(bf_buf, k0):
            for j in range(INNER):
                pltpu.async_copy(tbl.at[src_h.at[k0 + j]],
                                 bf_buf.at[pl.ds(j * CHUNK, CHUNK)], sem_g)

        def wait_gathers(bf_buf, k0):
            for j in range(INNER):
                pltpu.make_async_copy(
                    tbl.at[src_h.at[k0 + j]],
                    bf_buf.at[pl.ds(j * CHUNK, CHUNK)], sem_g).wait()

        CU = 4

        def convert(bf_buf, rows_buf):
            def cbody(i, carry):
                for u in range(CU):
                    r = i * CU + u
                    for k in range(SPL // 32):
                        v = bf_buf[r, pl.ds(k * 32, 32)]
                        rows_buf[r, pl.ds(k * 32, 16)] = (
                            v[0:16].astype(jnp.float32))
                        rows_buf[r, pl.ds(k * 32 + 16, 16)] = (
                            v[16:32].astype(jnp.float32))
                return carry
            lax.fori_loop(0, INNER * CHUNK // CU, cbody, 0)

        def fire_scatters(rows_buf, k0, deg_core):
            for j in range(INNER):
                pltpu.async_copy(rows_buf.at[pl.ds(j * CHUNK, CHUNK)],
                                 agg_sh.at[dst_h.at[k0 + j]], sem_s, add=True)
            if with_deg:
                @pl.when(c == deg_core)
                def _():
                    for j in range(INNER):
                        pltpu.async_copy(ones_v, deg_sh.at[dst_h.at[k0 + j]],
                                         sem_d, add=True)

        def wait_scatters(rows_buf, k0, deg_core):
            for j in range(INNER):
                pltpu.make_async_copy(
                    rows_buf.at[pl.ds(j * CHUNK, CHUNK)],
                    agg_sh.at[dst_h.at[k0 + j]], sem_s).wait()
            if with_deg:
                @pl.when(c == deg_core)
                def _():
                    for j in range(INNER):
                        pltpu.make_async_copy(
                            ones_v, deg_sh.at[dst_h.at[k0 + j]], sem_d).wait()

        for half in range(2):
            r0 = s * ROWS_PER_S + half * HROWS
            pltpu.sync_copy(src2.at[pl.ds(r0, HROWS)], src_h)
            pltpu.sync_copy(dst2.at[pl.ds(r0, HROWS)], dst_h)
            fire_gathers(bf0, 0)

            def tbody(t, carry):
                ka = 2 * t * INNER
                kb = ka + INNER

                @pl.when(t > 0)
                def _():
                    wait_scatters(rows1, ka - INNER, 1)
                fire_gathers(bf1, kb)
                wait_gathers(bf0, ka)
                convert(bf0, rows0)
                fire_scatters(rows0, ka, 0)
                wait_scatters(rows0, ka, 0)

                @pl.when(t < TSTEPS - 1)
                def _():
                    fire_gathers(bf0, kb + INNER)
                wait_gathers(bf1, kb)
                convert(bf1, rows1)
                fire_scatters(rows1, kb, 1)
                return carry

            lax.fori_loop(0, TSTEPS, tbody, 0)
            wait_scatters(rows1, (BLOCKS_H - 1) * INNER, 1)

        plsc.subcore_barrier()
        pltpu.sync_copy(agg_sh.at[pl.ds(s * RPS, RPS)],
                        out_agg.at[c, pl.ds(s * RPS, RPS)])
        if with_deg:
            pltpu.sync_copy(deg_sh.at[pl.ds(s * RPS, RPS)],
                            out_deg.at[c, pl.ds(s * RPS, RPS)])

    return pl.kernel(body, out_type=tuple(out_type) if with_deg else out_type[0],
                     mesh=mesh, scratch_types=scratch,
                     compiler_params=pltpu.CompilerParams(
                         use_tc_tiling_on_sc=False))


_SC_AGG_DEG = _make_sc_agg(True)
_SC_AGG = _make_sc_agg(False)


def _make_combine(relu, split_out):
    R = 1024
    G = NP // R

    def body(x_ref, a_ref, d0_ref, d1_ref, ws_ref, wn_ref, b_ref, o_ref):
        xcat = jnp.concatenate([x_ref[0], x_ref[1]], axis=1).astype(jnp.float32)
        deg = jnp.maximum(d0_ref[...] + d1_ref[...], 1.0)
        mean = jnp.concatenate([a_ref[0], a_ref[1]], axis=1) / deg
        y = (jnp.dot(xcat, ws_ref[...], preferred_element_type=jnp.float32)
             + jnp.dot(mean, wn_ref[...], preferred_element_type=jnp.float32)
             + b_ref[...])
        if relu:
            y = jnp.maximum(y, 0.0)
        if split_out:
            yb = y.astype(jnp.bfloat16)
            o_ref[0] = yb[:, :SPL]
            o_ref[1] = yb[:, SPL:]
        else:
            o_ref[...] = y

    if split_out:
        out_shape = jax.ShapeDtypeStruct((NC, NP, SPL), jnp.bfloat16)
        out_spec = pl.BlockSpec((NC, R, SPL), lambda i: (0, i, 0))
    else:
        out_shape = jax.ShapeDtypeStruct((NP, D), jnp.float32)
        out_spec = pl.BlockSpec((R, D), lambda i: (i, 0))

    return pl.pallas_call(
        body,
        grid=(G,),
        in_specs=[
            pl.BlockSpec((NC, R, SPL), lambda i: (0, i, 0)),
            pl.BlockSpec((NC, R, SPL), lambda i: (0, i, 0)),
            pl.BlockSpec((R, 1), lambda i: (i, 0)),
            pl.BlockSpec((R, 1), lambda i: (i, 0)),
            pl.BlockSpec((D, D), lambda i: (0, 0)),
            pl.BlockSpec((D, D), lambda i: (0, 0)),
            pl.BlockSpec((1, D), lambda i: (0, 0)),
        ],
        out_specs=out_spec,
        out_shape=out_shape,
    )


_COMBINE_RELU_SPLIT = _make_combine(True, True)
_COMBINE_PLAIN = _make_combine(False, False)


def kernel(x, edge_index, W1_self, W1_neigh, b1, W2_self, W2_neigh, b2):
    x = x.astype(jnp.float32)
    ei = edge_index.astype(jnp.int32)
    src2 = jnp.concatenate(
        [ei[0], jnp.zeros((EP - E,), jnp.int32)]).reshape(EP // CHUNK, CHUNK)
    dst2 = jnp.concatenate(
        [ei[1], jnp.full((EP - E,), NP - 1, jnp.int32)]).reshape(EP // CHUNK, CHUNK)
    xp = jnp.pad(x, ((0, NP - N), (0, 0)))
    xp3 = jnp.stack([xp[:, :SPL], xp[:, SPL:]]).astype(jnp.bfloat16)
    zrows = jnp.zeros((RPS, SPL), jnp.float32)
    zdeg = jnp.zeros((RPS,), jnp.float32)

    agg1, deg = _SC_AGG_DEG(xp3, src2, dst2, zrows, zdeg)
    d0 = deg[0][:, None]
    d1 = deg[1][:, None]
    h3 = _COMBINE_RELU_SPLIT(xp3, agg1, d0, d1, W1_self, W1_neigh,
                             b1.reshape(1, D))
    agg2 = _SC_AGG(h3, src2, dst2, zrows)
    out = _COMBINE_PLAIN(h3, agg2, d0, d1, W2_self, W2_neigh,
                         b2.reshape(1, D))
    return out[:N]


# speedup vs baseline: 1.0527x; 1.0527x over previous
import jax
import jax.numpy as jnp
from jax import lax
from jax.experimental import pallas as pl
from jax.experimental.pallas import tpu as pltpu
from jax.experimental.pallas import tpu_sc as plsc

N = 10000
E = 320000
D = 128
SPL = 64
NP = 10240
NC = 2
NS = 16
CHUNK = 128
INNER = 2
EP = 327680
ROWS_PER_S = EP // NS // CHUNK
HROWS = ROWS_PER_S // 2
BLOCKS_H = HROWS // INNER
TSTEPS = BLOCKS_H // 2
RPS = NP // NS


def _make_sc_agg(with_deg):
    out_type = [jax.ShapeDtypeStruct((NC, NP, SPL), jnp.float32)]
    scratch = [
        pltpu.VMEM((HROWS, CHUNK), jnp.int32),
        pltpu.VMEM((HROWS, CHUNK), jnp.int32),
        pltpu.VMEM((INNER * CHUNK, SPL), jnp.float32),
        pltpu.VMEM((INNER * CHUNK, SPL), jnp.float32),
        pltpu.VMEM((INNER * CHUNK, SPL), jnp.bfloat16),
        pltpu.VMEM((INNER * CHUNK, SPL), jnp.bfloat16),
        pltpu.VMEM_SHARED((NP, SPL), jnp.float32),
        pltpu.SemaphoreType.DMA,
        pltpu.SemaphoreType.DMA,
    ]
    if with_deg:
        out_type.append(jax.ShapeDtypeStruct((NC, NP), jnp.float32))
        scratch += [
            pltpu.VMEM((CHUNK,), jnp.float32),
            pltpu.VMEM_SHARED((NP,), jnp.float32),
            pltpu.SemaphoreType.DMA,
        ]

    mesh = plsc.VectorSubcoreMesh(core_axis_name="c", subcore_axis_name="s")

    def body(*refs):
        if with_deg:
            (table3, src2, dst2, zrows, zdeg, out_agg, out_deg,
             src_h, dst_h, rows0, rows1, bf0, bf1, agg_sh, sem_g, sem_s,
             ones_v, deg_sh, sem_d) = refs
        else:
            (table3, src2, dst2, zrows, out_agg,
             src_h, dst_h, rows0, rows1, bf0, bf1, agg_sh,
             sem_g, sem_s) = refs
        c = lax.axis_index("c")
        s = lax.axis_index("s")

        pltpu.sync_copy(zrows, agg_sh.at[pl.ds(s * RPS, RPS)])
        if with_deg:
            pltpu.sync_copy(zdeg, deg_sh.at[pl.ds(s * RPS, RPS)])
            for i in range(CHUNK // 16):
                ones_v[pl.ds(i * 16, 16)] = jnp.ones((16,), jnp.float32)
        plsc.subcore_barrier()

        tbl = table3.at[c]

        def fire_gathers(bf_buf, k0):
            for j in range(INNER):
                pltpu.async_copy(tbl.at[src_h.at[k0 + j]],
                                 bf_buf.at[pl.ds(j * CHUNK, CHUNK)], sem_g)

        def wait_gathers(bf_buf, k0):
            for j in range(INNER):
                pltpu.make_async_copy(
                    tbl.at[src_h.at[k0 + j]],
                    bf_buf.at[pl.ds(j * CHUNK, CHUNK)], sem_g).wait()

        CU = 8

        def convert(bf_buf, rows_buf):
            def cbody(i, carry):
                for u in range(CU):
                    r = i * CU + u
                    for k in range(SPL // 32):
                        v = bf_buf[r, pl.ds(k * 32, 32)]
                        rows_buf[r, pl.ds(k * 32, 16)] = (
                            v[0:16].astype(jnp.float32))
                        rows_buf[r, pl.ds(k * 32 + 16, 16)] = (
                            v[16:32].astype(jnp.float32))
                return carry
            lax.fori_loop(0, INNER * CHUNK // CU, cbody, 0)

        def fire_scatters(rows_buf, k0, deg_core):
            for j in range(INNER):
                pltpu.async_copy(rows_buf.at[pl.ds(j * CHUNK, CHUNK)],
                                 agg_sh.at[dst_h.at[k0 + j]], sem_s, add=True)
            if with_deg:
                @pl.when(c == deg_core)
                def _():
                    for j in range(INNER):
                        pltpu.async_copy(ones_v, deg_sh.at[dst_h.at[k0 + j]],
                                         sem_d, add=True)

        def wait_scatters(rows_buf, k0, deg_core):
            for j in range(INNER):
                pltpu.make_async_copy(
                    rows_buf.at[pl.ds(j * CHUNK, CHUNK)],
                    agg_sh.at[dst_h.at[k0 + j]], sem_s).wait()
            if with_deg:
                @pl.when(c == deg_core)
                def _():
                    for j in range(INNER):
                        pltpu.make_async_copy(
                            ones_v, deg_sh.at[dst_h.at[k0 + j]], sem_d).wait()

        for half in range(2):
            r0 = s * ROWS_PER_S + half * HROWS
            pltpu.sync_copy(src2.at[pl.ds(r0, HROWS)], src_h)
            pltpu.sync_copy(dst2.at[pl.ds(r0, HROWS)], dst_h)
            fire_gathers(bf0, 0)

            def tbody(t, carry):
                ka = 2 * t * INNER
                kb = ka + INNER

                @pl.when(t > 0)
                def _():
                    wait_scatters(rows1, ka - INNER, 1)
                fire_gathers(bf1, kb)
                wait_gathers(bf0, ka)

                @pl.when(t > 0)
                def _():
                    wait_scatters(rows0, ka - 2 * INNER, 0)
                convert(bf0, rows0)
                fire_scatters(rows0, ka, 0)

                @pl.when(t < TSTEPS - 1)
                def _():
                    fire_gathers(bf0, kb + INNER)
                wait_gathers(bf1, kb)
                convert(bf1, rows1)
                fire_scatters(rows1, kb, 1)
                return carry

            lax.fori_loop(0, TSTEPS, tbody, 0)
            wait_scatters(rows0, (BLOCKS_H - 2) * INNER, 0)
            wait_scatters(rows1, (BLOCKS_H - 1) * INNER, 1)

        plsc.subcore_barrier()
        pltpu.sync_copy(agg_sh.at[pl.ds(s * RPS, RPS)],
                        out_agg.at[c, pl.ds(s * RPS, RPS)])
        if with_deg:
            pltpu.sync_copy(deg_sh.at[pl.ds(s * RPS, RPS)],
                            out_deg.at[c, pl.ds(s * RPS, RPS)])

    return pl.kernel(body, out_type=tuple(out_type) if with_deg else out_type[0],
                     mesh=mesh, scratch_types=scratch,
                     compiler_params=pltpu.CompilerParams(
                         use_tc_tiling_on_sc=False))


_SC_AGG_DEG = _make_sc_agg(True)
_SC_AGG = _make_sc_agg(False)


def _make_combine(relu, split_out):
    R = 1024
    G = NP // R

    def body(x_ref, a_ref, d0_ref, d1_ref, ws_ref, wn_ref, b_ref, o_ref):
        xcat = jnp.concatenate([x_ref[0], x_ref[1]], axis=1).astype(jnp.float32)
        deg = jnp.maximum(d0_ref[...] + d1_ref[...], 1.0)
        mean = jnp.concatenate([a_ref[0], a_ref[1]], axis=1) / deg
        y = (jnp.dot(xcat, ws_ref[...], preferred_element_type=jnp.float32)
             + jnp.dot(mean, wn_ref[...], preferred_element_type=jnp.float32)
             + b_ref[...])
        if relu:
            y = jnp.maximum(y, 0.0)
        if split_out:
            yb = y.astype(jnp.bfloat16)
            o_ref[0] = yb[:, :SPL]
            o_ref[1] = yb[:, SPL:]
        else:
            o_ref[...] = y

    if split_out:
        out_shape = jax.ShapeDtypeStruct((NC, NP, SPL), jnp.bfloat16)
        out_spec = pl.BlockSpec((NC, R, SPL), lambda i: (0, i, 0))
    else:
        out_shape = jax.ShapeDtypeStruct((NP, D), jnp.float32)
        out_spec = pl.BlockSpec((R, D), lambda i: (i, 0))

    return pl.pallas_call(
        body,
        grid=(G,),
        in_specs=[
            pl.BlockSpec((NC, R, SPL), lambda i: (0, i, 0)),
            pl.BlockSpec((NC, R, SPL), lambda i: (0, i, 0)),
            pl.BlockSpec((R, 1), lambda i: (i, 0)),
            pl.BlockSpec((R, 1), lambda i: (i, 0)),
            pl.BlockSpec((D, D), lambda i: (0, 0)),
            pl.BlockSpec((D, D), lambda i: (0, 0)),
            pl.BlockSpec((1, D), lambda i: (0, 0)),
        ],
        out_specs=out_spec,
        out_shape=out_shape,
    )


_COMBINE_RELU_SPLIT = _make_combine(True, True)
_COMBINE_PLAIN = _make_combine(False, False)


def kernel(x, edge_index, W1_self, W1_neigh, b1, W2_self, W2_neigh, b2):
    x = x.astype(jnp.float32)
    ei = edge_index.astype(jnp.int32)
    src2 = jnp.concatenate(
        [ei[0], jnp.zeros((EP - E,), jnp.int32)]).reshape(EP // CHUNK, CHUNK)
    dst2 = jnp.concatenate(
        [ei[1], jnp.full((EP - E,), NP - 1, jnp.int32)]).reshape(EP // CHUNK, CHUNK)
    xp = jnp.pad(x, ((0, NP - N), (0, 0)))
    xp3 = jnp.stack([xp[:, :SPL], xp[:, SPL:]]).astype(jnp.bfloat16)
    zrows = jnp.zeros((RPS, SPL), jnp.float32)
    zdeg = jnp.zeros((RPS,), jnp.float32)

    agg1, deg = _SC_AGG_DEG(xp3, src2, dst2, zrows, zdeg)
    d0 = deg[0][:, None]
    d1 = deg[1][:, None]
    h3 = _COMBINE_RELU_SPLIT(xp3, agg1, d0, d1, W1_self, W1_neigh,
                             b1.reshape(1, D))
    agg2 = _SC_AGG(h3, src2, dst2, zrows)
    out = _COMBINE_PLAIN(h3, agg2, d0, d1, W2_self, W2_neigh,
                         b2.reshape(1, D))
    return out[:N]


# speedup vs baseline: 1.0800x; 1.0259x over previous
import jax
import jax.numpy as jnp
from jax import lax
from jax.experimental import pallas as pl
from jax.experimental.pallas import tpu as pltpu
from jax.experimental.pallas import tpu_sc as plsc

N = 10000
E = 320000
D = 128
SPL = 64
NP = 10240
NC = 2
NS = 16
CHUNK = 128
INNER = 2
EP = 327680
ROWS_PER_S = EP // NS // CHUNK
HROWS = ROWS_PER_S // 2
BLOCKS_H = HROWS // INNER
TSTEPS = BLOCKS_H // 2
RPS = NP // NS


def _make_sc_agg(with_deg):
    out_type = [jax.ShapeDtypeStruct((NC, NP, SPL), jnp.float32)]
    scratch = [
        pltpu.VMEM((HROWS, CHUNK), jnp.int32),
        pltpu.VMEM((HROWS, CHUNK), jnp.int32),
        pltpu.VMEM((INNER * CHUNK, SPL), jnp.float32),
        pltpu.VMEM((INNER * CHUNK, SPL), jnp.float32),
        pltpu.VMEM((INNER * CHUNK, SPL), jnp.bfloat16),
        pltpu.VMEM((INNER * CHUNK, SPL), jnp.bfloat16),
        pltpu.VMEM_SHARED((NP, SPL), jnp.float32),
        pltpu.SemaphoreType.DMA,
        pltpu.SemaphoreType.DMA,
    ]
    if with_deg:
        out_type.append(jax.ShapeDtypeStruct((NC, NP), jnp.float32))
        scratch += [
            pltpu.VMEM((CHUNK,), jnp.float32),
            pltpu.VMEM_SHARED((NP,), jnp.float32),
            pltpu.SemaphoreType.DMA,
        ]

    mesh = plsc.VectorSubcoreMesh(core_axis_name="c", subcore_axis_name="s")

    def body(*refs):
        if with_deg:
            (table3, src2, dst2, zrows, zdeg, out_agg, out_deg,
             src_h, dst_h, rows0, rows1, bf0, bf1, agg_sh, sem_g, sem_s,
             ones_v, deg_sh, sem_d) = refs
        else:
            (table3, src2, dst2, zrows, out_agg,
             src_h, dst_h, rows0, rows1, bf0, bf1, agg_sh,
             sem_g, sem_s) = refs
        c = lax.axis_index("c")
        s = lax.axis_index("s")

        pltpu.sync_copy(zrows, agg_sh.at[pl.ds(s * RPS, RPS)])
        if with_deg:
            pltpu.sync_copy(zdeg, deg_sh.at[pl.ds(s * RPS, RPS)])
            for i in range(CHUNK // 16):
                ones_v[pl.ds(i * 16, 16)] = jnp.ones((16,), jnp.float32)
        plsc.subcore_barrier()

        tbl = table3.at[c]

        def fire_gathers(bf_buf, k0):
            for j in range(INNER):
                pltpu.async_copy(tbl.at[src_h.at[k0 + j]],
                                 bf_buf.at[pl.ds(j * CHUNK, CHUNK)], sem_g)

        def wait_gathers(bf_buf, k0):
            for j in range(INNER):
                pltpu.make_async_copy(
                    tbl.at[src_h.at[k0 + j]],
                    bf_buf.at[pl.ds(j * CHUNK, CHUNK)], sem_g).wait()

        CU = 16

        def convert(bf_buf, rows_buf):
            def cbody(i, carry):
                for u in range(CU):
                    r = i * CU + u
                    for k in range(SPL // 32):
                        v = bf_buf[r, pl.ds(k * 32, 32)]
                        rows_buf[r, pl.ds(k * 32, 16)] = (
                            v[0:16].astype(jnp.float32))
                        rows_buf[r, pl.ds(k * 32 + 16, 16)] = (
                            v[16:32].astype(jnp.float32))
                return carry
            lax.fori_loop(0, INNER * CHUNK // CU, cbody, 0)

        def fire_scatters(rows_buf, k0, deg_core):
            for j in range(INNER):
                pltpu.async_copy(rows_buf.at[pl.ds(j * CHUNK, CHUNK)],
                                 agg_sh.at[dst_h.at[k0 + j]], sem_s, add=True)
            if with_deg:
                @pl.when(c == deg_core)
                def _():
                    for j in range(INNER):
                        pltpu.async_copy(ones_v, deg_sh.at[dst_h.at[k0 + j]],
                                         sem_d, add=True)

        def wait_scatters(rows_buf, k0, deg_core):
            for j in range(INNER):
                pltpu.make_async_copy(
                    rows_buf.at[pl.ds(j * CHUNK, CHUNK)],
                    agg_sh.at[dst_h.at[k0 + j]], sem_s).wait()
            if with_deg:
                @pl.when(c == deg_core)
                def _():
                    for j in range(INNER):
                        pltpu.make_async_copy(
                            ones_v, deg_sh.at[dst_h.at[k0 + j]], sem_d).wait()

        for half in range(2):
            r0 = s * ROWS_PER_S + half * HROWS
            pltpu.sync_copy(src2.at[pl.ds(r0, HROWS)], src_h)
            pltpu.sync_copy(dst2.at[pl.ds(r0, HROWS)], dst_h)
            fire_gathers(bf0, 0)

            def tbody(t, carry):
                ka = 2 * t * INNER
                kb = ka + INNER

                @pl.when(t > 0)
                def _():
                    wait_scatters(rows1, ka - INNER, 1)
                fire_gathers(bf1, kb)
                wait_gathers(bf0, ka)

                @pl.when(t > 0)
                def _():
                    wait_scatters(rows0, ka - 2 * INNER, 0)
                convert(bf0, rows0)
                fire_scatters(rows0, ka, 0)

                @pl.when(t < TSTEPS - 1)
                def _():
                    fire_gathers(bf0, kb + INNER)
                wait_gathers(bf1, kb)
                convert(bf1, rows1)
                fire_scatters(rows1, kb, 1)
                return carry

            lax.fori_loop(0, TSTEPS, tbody, 0)
            wait_scatters(rows0, (BLOCKS_H - 2) * INNER, 0)
            wait_scatters(rows1, (BLOCKS_H - 1) * INNER, 1)

        plsc.subcore_barrier()
        pltpu.sync_copy(agg_sh.at[pl.ds(s * RPS, RPS)],
                        out_agg.at[c, pl.ds(s * RPS, RPS)])
        if with_deg:
            pltpu.sync_copy(deg_sh.at[pl.ds(s * RPS, RPS)],
                            out_deg.at[c, pl.ds(s * RPS, RPS)])

    return pl.kernel(body, out_type=tuple(out_type) if with_deg else out_type[0],
                     mesh=mesh, scratch_types=scratch,
                     compiler_params=pltpu.CompilerParams(
                         use_tc_tiling_on_sc=False))


_SC_AGG_DEG = _make_sc_agg(True)
_SC_AGG = _make_sc_agg(False)


def _make_combine(relu, split_out):
    R = 1024
    G = NP // R

    def body(x_ref, a_ref, d0_ref, d1_ref, ws_ref, wn_ref, b_ref, o_ref):
        xcat = jnp.concatenate([x_ref[0], x_ref[1]], axis=1).astype(jnp.float32)
        deg = jnp.maximum(d0_ref[...] + d1_ref[...], 1.0)
        mean = jnp.concatenate([a_ref[0], a_ref[1]], axis=1) / deg
        y = (jnp.dot(xcat, ws_ref[...], preferred_element_type=jnp.float32)
             + jnp.dot(mean, wn_ref[...], preferred_element_type=jnp.float32)
             + b_ref[...])
        if relu:
            y = jnp.maximum(y, 0.0)
        if split_out:
            yb = y.astype(jnp.bfloat16)
            o_ref[0] = yb[:, :SPL]
            o_ref[1] = yb[:, SPL:]
        else:
            o_ref[...] = y

    if split_out:
        out_shape = jax.ShapeDtypeStruct((NC, NP, SPL), jnp.bfloat16)
        out_spec = pl.BlockSpec((NC, R, SPL), lambda i: (0, i, 0))
    else:
        out_shape = jax.ShapeDtypeStruct((NP, D), jnp.float32)
        out_spec = pl.BlockSpec((R, D), lambda i: (i, 0))

    return pl.pallas_call(
        body,
        grid=(G,),
        in_specs=[
            pl.BlockSpec((NC, R, SPL), lambda i: (0, i, 0)),
            pl.BlockSpec((NC, R, SPL), lambda i: (0, i, 0)),
            pl.BlockSpec((R, 1), lambda i: (i, 0)),
            pl.BlockSpec((R, 1), lambda i: (i, 0)),
            pl.BlockSpec((D, D), lambda i: (0, 0)),
            pl.BlockSpec((D, D), lambda i: (0, 0)),
            pl.BlockSpec((1, D), lambda i: (0, 0)),
        ],
        out_specs=out_spec,
        out_shape=out_shape,
    )


_COMBINE_RELU_SPLIT = _make_combine(True, True)
_COMBINE_PLAIN = _make_combine(False, False)


def kernel(x, edge_index, W1_self, W1_neigh, b1, W2_self, W2_neigh, b2):
    x = x.astype(jnp.float32)
    ei = edge_index.astype(jnp.int32)
    src2 = jnp.concatenate(
        [ei[0], jnp.zeros((EP - E,), jnp.int32)]).reshape(EP // CHUNK, CHUNK)
    dst2 = jnp.concatenate(
        [ei[1], jnp.full((EP - E,), NP - 1, jnp.int32)]).reshape(EP // CHUNK, CHUNK)
    xp = jnp.pad(x, ((0, NP - N), (0, 0)))
    xp3 = jnp.stack([xp[:, :SPL], xp[:, SPL:]]).astype(jnp.bfloat16)
    zrows = jnp.zeros((RPS, SPL), jnp.float32)
    zdeg = jnp.zeros((RPS,), jnp.float32)

    agg1, deg = _SC_AGG_DEG(xp3, src2, dst2, zrows, zdeg)
    d0 = deg[0][:, None]
    d1 = deg[1][:, None]
    h3 = _COMBINE_RELU_SPLIT(xp3, agg1, d0, d1, W1_self, W1_neigh,
                             b1.reshape(1, D))
    agg2 = _SC_AGG(h3, src2, dst2, zrows)
    out = _COMBINE_PLAIN(h3, agg2, d0, d1, W2_self, W2_neigh,
                         b2.reshape(1, D))
    return out[:N]
